# baseline (speedup 1.0000x reference)
import functools

import jax
import jax.numpy as jnp
from jax import lax
from jax.experimental import pallas as pl
from jax.experimental.pallas import tpu as pltpu
from jax.experimental.pallas import tpu_sc as plsc

_B, _T, _D, _H, _DFF, _V, _L = 2, 512, 1024, 16, 4096, 50257, 4
_DH = _D // _H
_N = _B * _T

_DFFB = 1024
_NF = _DFF // _DFFB
_VB = 2048
_NV = (_V + _VB - 1) // _VB

_F32 = jnp.float32


def _make_gather():
    nc, ns = 2, 16
    nw = nc * ns
    bpw = _N // nw
    mesh = plsc.VectorSubcoreMesh(
        core_axis_name="c", subcore_axis_name="s",
        num_cores=nc, num_subcores=ns)

    @functools.partial(
        pl.kernel,
        out_type=jax.ShapeDtypeStruct((_N, _D), _F32),
        mesh=mesh,
        scratch_types=[
            pltpu.VMEM((bpw,), jnp.int32),
            pltpu.VMEM((bpw, _D), _F32),
            pltpu.SemaphoreType.DMA,
        ],
    )
    def gather_k(ids_hbm, table_hbm, out_hbm, idx_v, rows_v, sem):
        wid = lax.axis_index("s") * nc + lax.axis_index("c")
        base = wid * bpw
        pltpu.sync_copy(ids_hbm.at[pl.ds(base, bpw)], idx_v)
        pltpu.async_copy(table_hbm.at[idx_v], rows_v, sem).wait()
        pltpu.sync_copy(rows_v, out_hbm.at[pl.ds(base, bpw)])

    return gather_k


_gather_fn = None


def _gather(ids, table):
    global _gather_fn
    if _gather_fn is None:
        _gather_fn = _make_gather()
    return _gather_fn(ids, table)


def _ln(x, g, b):
    mu = jnp.mean(x, axis=-1, keepdims=True)
    var = jnp.mean((x - mu) ** 2, axis=-1, keepdims=True)
    return (x - mu) * lax.rsqrt(var + 1e-5) * g + b


def _attn_body(first_layer):
    def body(*args):
        if first_layer:
            (x_ref, pos_ref, g_ref, b_ref, wq_ref, wk_ref, wv_ref, wo_ref,
             out_ref, q_s, k_s, v_s, o_s) = args
        else:
            (x_ref, g_ref, b_ref, wq_ref, wk_ref, wv_ref, wo_ref,
             out_ref, q_s, k_s, v_s, o_s) = args
        x = x_ref[...]
        if first_layer:
            x = (x.reshape(_B, _T, _D) + pos_ref[...][None]).reshape(_N, _D)
        out_ref[...] = x
        h = _ln(x, g_ref[0], b_ref[0])
        o_s[...] = h
        q_s[...] = jnp.dot(o_s[...], wq_ref[0], preferred_element_type=_F32)
        k_s[...] = jnp.dot(o_s[...], wk_ref[0], preferred_element_type=_F32)
        v_s[...] = jnp.dot(o_s[...], wv_ref[0], preferred_element_type=_F32)
        row = lax.broadcasted_iota(jnp.int32, (_T, _T), 0)
        col = lax.broadcasted_iota(jnp.int32, (_T, _T), 1)
        causal = col <= row
        scale = 1.0 / (_DH ** 0.5)
        for b in range(_B):
            for hh in range(_H):
                rs = pl.ds(b * _T, _T)
                cs = pl.ds(hh * _DH, _DH)
                qs = q_s[rs, cs].astype(jnp.bfloat16)
                ks = k_s[rs, cs].astype(jnp.bfloat16)
                vs = v_s[rs, cs].astype(jnp.bfloat16)
                att = lax.dot_general(
                    qs, ks, (((1,), (1,)), ((), ())),
                    preferred_element_type=_F32) * scale
                att = jnp.where(causal, att, _F32(-1e9))
                m = jnp.max(att, axis=-1, keepdims=True)
                e = jnp.exp(att - m)
                p = (e / jnp.sum(e, axis=-1, keepdims=True)).astype(jnp.bfloat16)
                o_s[rs, cs] = jnp.dot(p, vs, preferred_element_type=_F32)
        out_ref[...] += jnp.dot(o_s[...], wo_ref[0],
                                preferred_element_type=_F32)

    return body


def _attn_layer(l, x, pos_emb, ln1_g, ln1_b, wq, wk, wv, wo):
    first = l == 0
    in_specs = [pl.BlockSpec((_N, _D), lambda i: (0, 0))]
    operands = [x]
    if first:
        in_specs.append(pl.BlockSpec((_T, _D), lambda i: (0, 0)))
        operands.append(pos_emb)
    in_specs += [
        pl.BlockSpec((1, 1, _D), lambda i: (l, 0, 0)),
        pl.BlockSpec((1, 1, _D), lambda i: (l, 0, 0)),
        pl.BlockSpec((1, _D, _D), lambda i: (l, 0, 0)),
        pl.BlockSpec((1, _D, _D), lambda i: (l, 0, 0)),
        pl.BlockSpec((1, _D, _D), lambda i: (l, 0, 0)),
        pl.BlockSpec((1, _D, _D), lambda i: (l, 0, 0)),
    ]
    operands += [ln1_g.reshape(_L, 1, _D), ln1_b.reshape(_L, 1, _D),
                 wq, wk, wv, wo]
    return pl.pallas_call(
        _attn_body(first),
        grid=(1,),
        in_specs=in_specs,
        out_specs=pl.BlockSpec((_N, _D), lambda i: (0, 0)),
        out_shape=jax.ShapeDtypeStruct((_N, _D), _F32),
        scratch_shapes=[pltpu.VMEM((_N, _D), _F32)] * 4,
        compiler_params=pltpu.CompilerParams(
            vmem_limit_bytes=120 * 1024 * 1024),
    )(*operands)


def _ffn_body(final_ln):
    def body(*args):
        if final_ln:
            (x_ref, g_ref, b_ref, w1_ref, w3_ref, w2_ref, gf_ref, bf_ref,
             out_ref, hi_s, lo_s) = args
        else:
            (x_ref, g_ref, b_ref, w1_ref, w3_ref, w2_ref,
             out_ref, hi_s, lo_s) = args
        f = pl.program_id(0)

        @pl.when(f == 0)
        def _():
            x = x_ref[...]
            out_ref[...] = x
            h2 = _ln(x, g_ref[0], b_ref[0])
            hi = h2.astype(jnp.bfloat16)
            hi_s[...] = hi
            lo_s[...] = (h2 - hi.astype(_F32)).astype(jnp.bfloat16)

        w1b = w1_ref[0].astype(jnp.bfloat16)
        w3b = w3_ref[0].astype(jnp.bfloat16)
        u = (jnp.dot(hi_s[...], w1b, preferred_element_type=_F32)
             + jnp.dot(lo_s[...], w1b, preferred_element_type=_F32))
        g = (jnp.dot(hi_s[...], w3b, preferred_element_type=_F32)
             + jnp.dot(lo_s[...], w3b, preferred_element_type=_F32))
        a = (u / (1.0 + jnp.exp(-u))) * g
        out_ref[...] += jnp.dot(a, w2_ref[0], preferred_element_type=_F32)

        if final_ln:
            @pl.when(f == _NF - 1)
            def _():
                out_ref[...] = _ln(out_ref[...], gf_ref[...], bf_ref[...])

    return body


def _ffn_layer(l, x, ln2_g, ln2_b, w1, w3, w2, lnf_g, lnf_b):
    final = l == _L - 1
    in_specs = [
        pl.BlockSpec((_N, _D), lambda f: (0, 0)),
        pl.BlockSpec((1, 1, _D), lambda f: (l, 0, 0)),
        pl.BlockSpec((1, 1, _D), lambda f: (l, 0, 0)),
        pl.BlockSpec((1, _D, _DFFB), lambda f: (l, 0, f)),
        pl.BlockSpec((1, _D, _DFFB), lambda f: (l, 0, f)),
        pl.BlockSpec((1, _DFFB, _D), lambda f: (l, f, 0)),
    ]
    operands = [x, ln2_g.reshape(_L, 1, _D), ln2_b.reshape(_L, 1, _D),
                w1, w3, w2]
    if final:
        in_specs += [
            pl.BlockSpec((1, _D), lambda f: (0, 0)),
            pl.BlockSpec((1, _D), lambda f: (0, 0)),
        ]
        operands += [lnf_g.reshape(1, _D), lnf_b.reshape(1, _D)]
    return pl.pallas_call(
        _ffn_body(final),
        grid=(_NF,),
        in_specs=in_specs,
        out_specs=pl.BlockSpec((_N, _D), lambda f: (0, 0)),
        out_shape=jax.ShapeDtypeStruct((_N, _D), _F32),
        scratch_shapes=[pltpu.VMEM((_N, _D), jnp.bfloat16)] * 2,
        compiler_params=pltpu.CompilerParams(
            vmem_limit_bytes=120 * 1024 * 1024),
    )(*operands)


def _head_body(xf_ref, wh_ref, out_ref):
    a = xf_ref[...].astype(jnp.bfloat16)
    w = wh_ref[...].astype(jnp.bfloat16)
    out_ref[...] = lax.dot_general(
        a, w, (((1,), (1,)), ((), ())), preferred_element_type=_F32)


def _head(xf, w_head):
    return pl.pallas_call(
        _head_body,
        grid=(_NV,),
        in_specs=[
            pl.BlockSpec((_N, _D), lambda i: (0, 0)),
            pl.BlockSpec((_VB, _D), lambda i: (i, 0)),
        ],
        out_specs=pl.BlockSpec((_N, _VB), lambda i: (0, i)),
        out_shape=jax.ShapeDtypeStruct((_N, _V), _F32),
        compiler_params=pltpu.CompilerParams(
            vmem_limit_bytes=120 * 1024 * 1024),
    )(xf, w_head)


def kernel(input_ids, tok_emb, pos_emb, ln1_g, ln1_b, wq, wk, wv, wo,
           ln2_g, ln2_b, w1, w3, w2, lnf_g, lnf_b, w_head):
    ids = input_ids.reshape(_N).astype(jnp.int32)
    x = _gather(ids, tok_emb)
    for l in range(_L):
        x = _attn_layer(l, x, pos_emb, ln1_g, ln1_b, wq, wk, wv, wo)
        x = _ffn_layer(l, x, ln2_g, ln2_b, w1, w3, w2, lnf_g, lnf_b)
    logits = _head(x, w_head)
    return logits.reshape(_B, _T, _V)


# speedup vs baseline: 1.4578x; 1.4578x over previous
import functools

import jax
import jax.numpy as jnp
from jax import lax
from jax.experimental import pallas as pl
from jax.experimental.pallas import tpu as pltpu
from jax.experimental.pallas import tpu_sc as plsc

_B, _T, _D, _H, _DFF, _V, _L = 2, 512, 1024, 16, 4096, 50257, 4
_DH = _D // _H
_N = _B * _T

_DFFB = 1024
_NF = _DFF // _DFFB
_VB = 2048
_NV = (_V + _VB - 1) // _VB

_F32 = jnp.float32


def _make_gather():
    nc, ns = 2, 16
    nw = nc * ns
    bpw = _N // nw
    mesh = plsc.VectorSubcoreMesh(
        core_axis_name="c", subcore_axis_name="s",
        num_cores=nc, num_subcores=ns)

    @functools.partial(
        pl.kernel,
        out_type=jax.ShapeDtypeStruct((_N, _D), _F32),
        mesh=mesh,
        scratch_types=[
            pltpu.VMEM((bpw,), jnp.int32),
            pltpu.VMEM((bpw, _D), _F32),
            pltpu.SemaphoreType.DMA,
        ],
    )
    def gather_k(ids_hbm, table_hbm, out_hbm, idx_v, rows_v, sem):
        wid = lax.axis_index("s") * nc + lax.axis_index("c")
        base = wid * bpw
        pltpu.sync_copy(ids_hbm.at[pl.ds(base, bpw)], idx_v)
        pltpu.async_copy(table_hbm.at[idx_v], rows_v, sem).wait()
        pltpu.sync_copy(rows_v, out_hbm.at[pl.ds(base, bpw)])

    return gather_k


_gather_fn = None


def _gather(ids, table):
    global _gather_fn
    if _gather_fn is None:
        _gather_fn = _make_gather()
    return _gather_fn(ids, table)


def _ln(x, g, b):
    mu = jnp.mean(x, axis=-1, keepdims=True)
    var = jnp.mean((x - mu) ** 2, axis=-1, keepdims=True)
    return (x - mu) * lax.rsqrt(var + 1e-5) * g + b


def _attn_body(first_layer):
    def body(*args):
        if first_layer:
            (x_ref, pos_ref, g_ref, b_ref, wq_ref, wk_ref, wv_ref, wo_ref,
             out_ref, q_s, k_s, v_s, o_s) = args
        else:
            (x_ref, g_ref, b_ref, wq_ref, wk_ref, wv_ref, wo_ref,
             out_ref, q_s, k_s, v_s, o_s) = args
        x = x_ref[...]
        if first_layer:
            x = (x.reshape(_B, _T, _D) + pos_ref[...][None]).reshape(_N, _D)
        out_ref[...] = x
        h = _ln(x, g_ref[0], b_ref[0])
        o_s[...] = h
        q_s[...] = jnp.dot(o_s[...], wq_ref[0], preferred_element_type=_F32)
        k_s[...] = jnp.dot(o_s[...], wk_ref[0], preferred_element_type=_F32)
        v_s[...] = jnp.dot(o_s[...], wv_ref[0], preferred_element_type=_F32)
        row = lax.broadcasted_iota(jnp.int32, (_T, _T), 0)
        col = lax.broadcasted_iota(jnp.int32, (_T, _T), 1)
        causal = col <= row
        scale = 1.0 / (_DH ** 0.5)
        for b in range(_B):
            for hh in range(_H):
                rs = pl.ds(b * _T, _T)
                cs = pl.ds(hh * _DH, _DH)
                qs = q_s[rs, cs]
                ks = k_s[rs, cs]
                vs = v_s[rs, cs]
                att = lax.dot_general(
                    qs, ks, (((1,), (1,)), ((), ())),
                    preferred_element_type=_F32) * scale
                att = jnp.where(causal, att, _F32(-1e9))
                m = jnp.max(att, axis=-1, keepdims=True)
                e = jnp.exp(att - m)
                p = e / jnp.sum(e, axis=-1, keepdims=True)
                o_s[rs, cs] = jnp.dot(p, vs, preferred_element_type=_F32)
        out_ref[...] += jnp.dot(o_s[...], wo_ref[0],
                                preferred_element_type=_F32)

    return body


def _attn_layer(l, x, pos_emb, ln1_g, ln1_b, wq, wk, wv, wo):
    first = l == 0
    in_specs = [pl.BlockSpec((_N, _D), lambda i: (0, 0))]
    operands = [x]
    if first:
        in_specs.append(pl.BlockSpec((_T, _D), lambda i: (0, 0)))
        operands.append(pos_emb)
    in_specs += [
        pl.BlockSpec((1, 1, _D), lambda i: (l, 0, 0)),
        pl.BlockSpec((1, 1, _D), lambda i: (l, 0, 0)),
        pl.BlockSpec((1, _D, _D), lambda i: (l, 0, 0)),
        pl.BlockSpec((1, _D, _D), lambda i: (l, 0, 0)),
        pl.BlockSpec((1, _D, _D), lambda i: (l, 0, 0)),
        pl.BlockSpec((1, _D, _D), lambda i: (l, 0, 0)),
    ]
    operands += [ln1_g.reshape(_L, 1, _D), ln1_b.reshape(_L, 1, _D),
                 wq, wk, wv, wo]
    return pl.pallas_call(
        _attn_body(first),
        grid=(1,),
        in_specs=in_specs,
        out_specs=pl.BlockSpec((_N, _D), lambda i: (0, 0)),
        out_shape=jax.ShapeDtypeStruct((_N, _D), _F32),
        scratch_shapes=[pltpu.VMEM((_N, _D), _F32)] * 4,
        compiler_params=pltpu.CompilerParams(
            vmem_limit_bytes=120 * 1024 * 1024),
    )(*operands)


def _ffn_body(final_ln):
    def body(*args):
        if final_ln:
            (x_ref, g_ref, b_ref, w1_ref, w3_ref, w2_ref, gf_ref, bf_ref,
             out_ref, h2_s) = args
        else:
            (x_ref, g_ref, b_ref, w1_ref, w3_ref, w2_ref,
             out_ref, h2_s) = args
        f = pl.program_id(0)

        @pl.when(f == 0)
        def _():
            x = x_ref[...]
            out_ref[...] = x
            h2_s[...] = _ln(x, g_ref[0], b_ref[0])

        u = jnp.dot(h2_s[...], w1_ref[0], preferred_element_type=_F32)
        g = jnp.dot(h2_s[...], w3_ref[0], preferred_element_type=_F32)
        a = (u / (1.0 + jnp.exp(-u))) * g
        out_ref[...] += jnp.dot(a, w2_ref[0], preferred_element_type=_F32)

        if final_ln:
            @pl.when(f == _NF - 1)
            def _():
                y = _ln(out_ref[...], gf_ref[...], bf_ref[...])
                for b in range(_B):
                    for t4 in range(_T // 128):
                        dst = pl.ds((t4 * _B + b) * 128, 128)
                        out_ref[dst, :] = y[b * _T + t4 * 128:
                                            b * _T + t4 * 128 + 128, :]

    return body


def _ffn_layer(l, x, ln2_g, ln2_b, w1, w3, w2, lnf_g, lnf_b):
    final = l == _L - 1
    in_specs = [
        pl.BlockSpec((_N, _D), lambda f: (0, 0)),
        pl.BlockSpec((1, 1, _D), lambda f: (l, 0, 0)),
        pl.BlockSpec((1, 1, _D), lambda f: (l, 0, 0)),
        pl.BlockSpec((1, _D, _DFFB), lambda f: (l, 0, f)),
        pl.BlockSpec((1, _D, _DFFB), lambda f: (l, 0, f)),
        pl.BlockSpec((1, _DFFB, _D), lambda f: (l, f, 0)),
    ]
    operands = [x, ln2_g.reshape(_L, 1, _D), ln2_b.reshape(_L, 1, _D),
                w1, w3, w2]
    if final:
        in_specs += [
            pl.BlockSpec((1, _D), lambda f: (0, 0)),
            pl.BlockSpec((1, _D), lambda f: (0, 0)),
        ]
        operands += [lnf_g.reshape(1, _D), lnf_b.reshape(1, _D)]
    return pl.pallas_call(
        _ffn_body(final),
        grid=(_NF,),
        in_specs=in_specs,
        out_specs=pl.BlockSpec((_N, _D), lambda f: (0, 0)),
        out_shape=jax.ShapeDtypeStruct((_N, _D), _F32),
        scratch_shapes=[pltpu.VMEM((_N, _D), _F32)],
        compiler_params=pltpu.CompilerParams(
            vmem_limit_bytes=120 * 1024 * 1024),
    )(*operands)


def _head_body(xf_ref, wh_ref, out_ref):
    a = xf_ref[...].astype(jnp.bfloat16)
    w = wh_ref[...].astype(jnp.bfloat16)
    r = lax.dot_general(
        w, a, (((1,), (1,)), ((), ())), preferred_element_type=_F32)
    out_ref[...] = r.reshape(_VB, _T // 128, _B, 128)


def _head(xf, w_head):
    return pl.pallas_call(
        _head_body,
        grid=(_NV,),
        in_specs=[
            pl.BlockSpec((_N, _D), lambda i: (0, 0)),
            pl.BlockSpec((_VB, _D), lambda i: (i, 0)),
        ],
        out_specs=pl.BlockSpec((_VB, _T // 128, _B, 128),
                               lambda i: (i, 0, 0, 0)),
        out_shape=jax.ShapeDtypeStruct((_V, _T // 128, _B, 128), _F32),
        compiler_params=pltpu.CompilerParams(
            vmem_limit_bytes=120 * 1024 * 1024),
    )(xf, w_head)


def kernel(input_ids, tok_emb, pos_emb, ln1_g, ln1_b, wq, wk, wv, wo,
           ln2_g, ln2_b, w1, w3, w2, lnf_g, lnf_b, w_head):
    ids = input_ids.reshape(_N).astype(jnp.int32)
    x = _gather(ids, tok_emb)
    for l in range(_L):
        x = _attn_layer(l, x, pos_emb, ln1_g, ln1_b, wq, wk, wv, wo)
        x = _ffn_layer(l, x, ln2_g, ln2_b, w1, w3, w2, lnf_g, lnf_b)
    logits = _head(x, w_head)
    return jnp.transpose(logits, (2, 1, 3, 0)).reshape(_B, _T, _V)
